# baseline (speedup 1.0000x reference)
import functools

import jax
import jax.numpy as jnp
from jax import lax
from jax.experimental import pallas as pl
from jax.experimental.pallas import tpu as pltpu
from jax.experimental.pallas import tpu_sc as plsc

N = 10000
E = 320000
D = 128
G = 64
NC = 2
NS = 16
NW = NC * NS
EPT = E // NW
CHUNK = 128
NCHUNK = 80
EPT_P = NCHUNK * CHUNK
N_PAD = N + 8
RPT = N // NS
BN_INV = 1.0 / (1.0 + 1e-5) ** 0.5


def _sc_agg_body(h_hbm, src_hbm, dst_hbm, zeros_hbm, out_hbm,
                 agg_sh, dst_v, sb0, sb1, rows0, rows1,
                 sem0, sem1, si0, si1):
    c = lax.axis_index("c")
    s = lax.axis_index("s")
    pltpu.sync_copy(zeros_hbm, agg_sh.at[pl.ds(s * RPT, RPT)])
    pltpu.sync_copy(dst_hbm.at[c, s], dst_v)

    def sldx(i, sb, si):
        pltpu.async_copy(src_hbm.at[c, s, pl.ds(i, 1)], sb, si)

    def iwait(i, sb, si):
        pltpu.make_async_copy(src_hbm.at[c, s, pl.ds(i, 1)], sb, si).wait()

    def gather(sb, buf, sem):
        pltpu.async_copy(h_hbm.at[sb.at[0]], buf, sem)

    def gwait(sb, buf, sem):
        pltpu.make_async_copy(h_hbm.at[sb.at[0]], buf, sem).wait()

    def scat(i, buf):
        pltpu.sync_copy(buf, agg_sh.at[dst_v.at[i]], add=True)

    plsc.subcore_barrier()

    def halfstep(i, ib_c, ib_n, r_c, r_n, sem_c, sem_n, sic, sin):
        gwait(ib_c, r_c, sem_c)

        @pl.when(i + 2 < NCHUNK)
        def _():
            sldx(i + 2, ib_c, sic)

        @pl.when(i + 1 < NCHUNK)
        def _():
            iwait(i + 1, ib_n, sin)
            gather(ib_n, r_n, sem_n)

        scat(i, r_c)

    sldx(0, sb0, si0)
    iwait(0, sb0, si0)
    gather(sb0, rows0, sem0)
    sldx(1, sb1, si1)

    def pair(j, carry):
        i0 = 2 * j
        halfstep(i0, sb0, sb1, rows0, rows1, sem0, sem1, si0, si1)
        halfstep(i0 + 1, sb1, sb0, rows1, rows0, sem1, sem0, si1, si0)
        return carry

    lax.fori_loop(0, NCHUNK // 2, pair, 0)
    plsc.subcore_barrier()
    pltpu.sync_copy(agg_sh.at[pl.ds(s * RPT, RPT)], out_hbm.at[c, s])


_sc_agg = pl.kernel(
    _sc_agg_body,
    out_type=jax.ShapeDtypeStruct((NC, NS, RPT, D), jnp.float32),
    mesh=plsc.VectorSubcoreMesh(core_axis_name="c", subcore_axis_name="s"),
    scratch_types=[
        pltpu.VMEM_SHARED((N_PAD, D), jnp.float32),
        pltpu.VMEM((NCHUNK, CHUNK), jnp.int32),
        pltpu.VMEM((1, CHUNK), jnp.int32),
        pltpu.VMEM((1, CHUNK), jnp.int32),
        pltpu.VMEM((CHUNK, D), jnp.float32),
        pltpu.VMEM((CHUNK, D), jnp.float32),
        pltpu.SemaphoreType.DMA,
        pltpu.SemaphoreType.DMA,
        pltpu.SemaphoreType.DMA,
        pltpu.SemaphoreType.DMA,
    ],
)


def _tc_layer_body(h_ref, p0_ref, p1_ref, w1_ref, b1_ref, g_ref, be_ref,
                   w2_ref, b2_ref, o_ref):
    z = h_ref[...] + p0_ref[...] + p1_ref[...]
    z = jnp.dot(z, w1_ref[...], preferred_element_type=jnp.float32)
    z = (z + b1_ref[...]) * (g_ref[...] * BN_INV) + be_ref[...]
    z = jnp.maximum(z, 0.0)
    z = jnp.dot(z, w2_ref[...], preferred_element_type=jnp.float32)
    o_ref[...] = jnp.maximum(z + b2_ref[...], 0.0)


def _tc_layer(h, p0, p1, w1, b1, g, be, w2, b2):
    nb = 10
    blk = N // nb
    row_spec = pl.BlockSpec((blk, D), lambda i: (i, 0))
    full = pl.BlockSpec((D, D), lambda i: (0, 0))
    vec = pl.BlockSpec((1, D), lambda i: (0, 0))
    return pl.pallas_call(
        _tc_layer_body,
        grid=(nb,),
        in_specs=[row_spec, row_spec, row_spec, full, vec, vec, vec, full, vec],
        out_specs=row_spec,
        out_shape=jax.ShapeDtypeStruct((N, D), jnp.float32),
    )(h, p0, p1, w1, b1.reshape(1, D), g.reshape(1, D), be.reshape(1, D),
      w2, b2.reshape(1, D))


def _tc_pool_head_body(h_ref, batch_ref, w1_ref, b1_ref, w2_ref, b2_ref, o_ref):
    gids = lax.broadcasted_iota(jnp.int32, (G, N), 0)
    onehot = (batch_ref[...] == gids).astype(jnp.float32)
    sums = jnp.dot(onehot, h_ref[...], preferred_element_type=jnp.float32)
    cnts = jnp.sum(onehot, axis=1, keepdims=True)
    pooled = sums / jnp.maximum(cnts, 1.0)
    z = jnp.dot(pooled, w1_ref[...], preferred_element_type=jnp.float32)
    z = jnp.maximum(z + b1_ref[...], 0.0)
    z = jnp.dot(z, w2_ref[...], preferred_element_type=jnp.float32)
    o_ref[...] = z + b2_ref[...]


def _tc_pool_head(h, batch, w1, b1, w2, b2):
    return pl.pallas_call(
        _tc_pool_head_body,
        out_shape=jax.ShapeDtypeStruct((G, 10), jnp.float32),
    )(h, batch.reshape(1, N), w1, b1.reshape(1, D), w2, b2.reshape(1, 10))


@jax.jit
def kernel(x, edge_index, batch, conv_W1, conv_b1, conv_gamma, conv_beta,
           conv_W2, conv_b2, head_W1, head_b1, head_W2, head_b2):
    pad = EPT_P - EPT
    src = jnp.pad(edge_index[0].reshape(NW, EPT), ((0, 0), (0, pad)))
    dst = jnp.pad(edge_index[1].reshape(NW, EPT), ((0, 0), (0, pad)),
                  constant_values=N)
    src = src.reshape(NC, NS, NCHUNK, CHUNK)
    dst = dst.reshape(NC, NS, NCHUNK, CHUNK)
    zeros = jnp.zeros((RPT, D), dtype=jnp.float32)
    h = x
    for i in range(3):
        p = _sc_agg(h, src, dst, zeros).reshape(NC, N, D)
        h = _tc_layer(h, p[0], p[1], conv_W1[i], conv_b1[i], conv_gamma[i],
                      conv_beta[i], conv_W2[i], conv_b2[i])
    return _tc_pool_head(h, batch, head_W1, head_b1, head_W2, head_b2)
